# Initial kernel scaffold; baseline (speedup 1.0000x reference)
import jax
import jax.numpy as jnp
from jax.experimental import pallas as pl


def kernel(x, edge_index, Wc1, bc1, Wc2, bc2, Wc3, bc3, Wc4, bc4, Wl1, bl1, Wl2, bl2, Wl3, bl3, Wm1, bm1, Wm2, bm2, Wm3, bm3, Wm4, bm4, Wcls, bcls):
    raise NotImplementedError("write your pallas kernel here")


# speedup vs baseline: 4.5536x; 4.5536x over previous
import functools

import jax
import jax.numpy as jnp
from jax import lax
from jax.experimental import pallas as pl
from jax.experimental.pallas import tpu as pltpu
from jax.experimental.pallas import tpu_sc as plsc

LANES = 16
NSUB = 16
NCORE = 2
NTILE = NCORE * NSUB
EBLK = 128

_mesh = lambda: plsc.VectorSubcoreMesh(
    core_axis_name="c", subcore_axis_name="s", num_cores=NCORE, num_subcores=NSUB)


def _fill(ref, nrows, value):
    val = jnp.full((LANES,), value, jnp.float32)

    def body(i, _):
        ref[i, :] = val
        return 0

    lax.fori_loop(0, nrows, body, 0)


def _zero_own_rows(acc, zbuf, sid, rows_per_sub):
    base = sid * rows_per_sub
    nfull = rows_per_sub // 1024
    rem = rows_per_sub - nfull * 1024
    for t in range(nfull):
        pltpu.sync_copy(zbuf, acc.at[pl.ds(base + t * 1024, 1024)])
    if rem:
        pltpu.sync_copy(zbuf.at[pl.ds(0, rem)], acc.at[pl.ds(base + nfull * 1024, rem)])


def _scatter_chunk(z_ref, row_v, col_v, gbuf, gsems, acc, nblk2):
    pltpu.make_async_copy(z_ref.at[row_v.at[0]], gbuf.at[0], gsems.at[0]).start()

    def step(j, _):
        b = lax.rem(j, 2)
        pb = 1 - b
        pltpu.make_async_copy(z_ref.at[row_v.at[j]], gbuf.at[b], gsems.at[b]).start()
        pltpu.make_async_copy(z_ref.at[pl.ds(0, EBLK)], gbuf.at[pb], gsems.at[pb]).wait()
        pltpu.sync_copy(gbuf.at[pb], acc.at[col_v.at[j - 1]], add=True)
        return 0

    lax.fori_loop(1, nblk2, step, 0)
    bl = (nblk2 - 1) % 2
    pltpu.make_async_copy(z_ref.at[pl.ds(0, EBLK)], gbuf.at[bl], gsems.at[bl]).wait()
    pltpu.sync_copy(gbuf.at[bl], acc.at[col_v.at[nblk2 - 1]], add=True)


def _writeback(acc, out_ref, k, sid, rows_per_sub):
    base = sid * rows_per_sub
    pltpu.sync_copy(acc.at[pl.ds(base, rows_per_sub)],
                    out_ref.at[k, pl.ds(base, rows_per_sub)])


def _make_deg_kernel(n8, nblk):
    rows_per_sub = n8 // NSUB

    @functools.partial(
        pl.kernel,
        out_type=jax.ShapeDtypeStruct((NCORE, n8, LANES), jnp.float32),
        mesh=_mesh(),
        compiler_params=pltpu.CompilerParams(use_tc_tiling_on_sc=False),
        scratch_types=[
            pltpu.VMEM((nblk, EBLK), jnp.int32),
            pltpu.VMEM((EBLK, LANES), jnp.float32),
            pltpu.VMEM((1024, LANES), jnp.float32),
            pltpu.VMEM_SHARED((n8, LANES), jnp.float32),
        ],
    )
    def deg_kernel(col_hbm, out_ref, col_v, obuf, zbuf, acc):
        cid = lax.axis_index("c")
        sid = lax.axis_index("s")
        wid = sid * NCORE + cid
        pltpu.sync_copy(col_hbm.at[wid], col_v)
        _fill(obuf, EBLK, 1.0)
        _fill(zbuf, 1024, 0.0)
        _zero_own_rows(acc, zbuf, sid, rows_per_sub)
        plsc.subcore_barrier()

        def body(j, _):
            pltpu.sync_copy(obuf, acc.at[col_v.at[j]], add=True)
            return 0

        lax.fori_loop(0, nblk, body, 0)
        plsc.subcore_barrier()

        @pl.when(cid == 0)
        def _():
            _writeback(acc, out_ref, 0, sid, rows_per_sub)

        @pl.when(cid == 1)
        def _():
            _writeback(acc, out_ref, 1, sid, rows_per_sub)

    return deg_kernel


def _make_scatter_kernel(nc, n8, nblk):
    rows_per_sub = n8 // NSUB
    nout = nc if nc > 1 else 2

    @functools.partial(
        pl.kernel,
        out_type=jax.ShapeDtypeStruct((nout, n8, LANES), jnp.float32),
        mesh=_mesh(),
        compiler_params=pltpu.CompilerParams(use_tc_tiling_on_sc=False),
        scratch_types=[
            pltpu.VMEM((nblk, EBLK), jnp.int32),
            pltpu.VMEM((nblk, EBLK), jnp.int32),
            pltpu.VMEM((2, EBLK, LANES), jnp.float32),
            pltpu.VMEM((1024, LANES), jnp.float32),
            pltpu.VMEM_SHARED((n8, LANES), jnp.float32),
            pltpu.SemaphoreType.DMA((2,)),
        ],
    )
    def scatter_kernel(*refs):
        z_refs = refs[:nc]
        row_hbm, col_hbm, out_ref = refs[nc], refs[nc + 1], refs[nc + 2]
        row_v, col_v, gbuf, zbuf, acc, gsems = refs[nc + 3:]
        cid = lax.axis_index("c")
        sid = lax.axis_index("s")
        _fill(zbuf, 1024, 0.0)
        wid = sid * NCORE + cid

        def load_idx(slot):
            pltpu.sync_copy(row_hbm.at[slot], row_v)
            pltpu.sync_copy(col_hbm.at[slot], col_v)

        def process(z_ref, k):
            _zero_own_rows(acc, zbuf, sid, rows_per_sub)
            plsc.subcore_barrier()
            if nc == 1:
                load_idx(wid)
                _scatter_chunk(z_ref, row_v, col_v, gbuf, gsems, acc, nblk)
            else:
                for half in range(2):
                    load_idx(2 * sid + half)
                    _scatter_chunk(z_ref, row_v, col_v, gbuf, gsems, acc, nblk)
            plsc.subcore_barrier()
            _writeback(acc, out_ref, k, sid, rows_per_sub)
            plsc.subcore_barrier()

        if nc == 4:
            @pl.when(cid == 0)
            def _():
                process(z_refs[0], 0)
                process(z_refs[1], 1)

            @pl.when(cid == 1)
            def _():
                process(z_refs[2], 2)
                process(z_refs[3], 3)
        elif nc == 2:
            @pl.when(cid == 0)
            def _():
                process(z_refs[0], 0)

            @pl.when(cid == 1)
            def _():
                process(z_refs[1], 1)
        else:
            @pl.when(cid == 0)
            def _():
                process(z_refs[0], 0)

            @pl.when(cid == 1)
            def _():
                process(z_refs[0], 1)

    return scatter_kernel


def _make_edge_out_kernel(n, n8, nblk):
    nbuf = n8

    @functools.partial(
        pl.kernel,
        out_type=jax.ShapeDtypeStruct((NTILE, nblk, EBLK), jnp.float32),
        mesh=_mesh(),
        compiler_params=pltpu.CompilerParams(
            use_tc_tiling_on_sc=False, needs_layout_passes=False),
        scratch_types=[
            pltpu.VMEM((nbuf,), jnp.float32),
            pltpu.VMEM((nblk, EBLK), jnp.int32),
            pltpu.VMEM((nblk, EBLK), jnp.int32),
            pltpu.VMEM((nblk, EBLK), jnp.float32),
            pltpu.VMEM((nblk, EBLK), jnp.float32),
        ],
    )
    def edge_out_kernel(u_hbm, row_hbm, col_hbm, v_hbm, out_ref,
                        u_v, row_v, col_v, v_v, out_v):
        cid = lax.axis_index("c")
        sid = lax.axis_index("s")
        wid = sid * NCORE + cid
        pltpu.sync_copy(u_hbm, u_v.at[pl.ds(0, n)])
        pltpu.sync_copy(row_hbm.at[wid], row_v)
        pltpu.sync_copy(col_hbm.at[wid], col_v)
        pltpu.sync_copy(v_hbm.at[wid], v_v)

        def body(j, _):
            for t in range(EBLK // LANES):
                sl = pl.ds(t * LANES, LANES)
                ridx = row_v[j, sl]
                cidx = col_v[j, sl]
                g1 = plsc.load_gather(u_v, [ridx])
                g2 = plsc.load_gather(u_v, [cidx])
                out_v[j, sl] = g1 + g2 + v_v[j, sl]
            return 0

        lax.fori_loop(0, nblk, body, 0)
        pltpu.sync_copy(out_v, out_ref.at[wid])

    return edge_out_kernel


_R = 2000


def _tc_specs(shapes_and_maps):
    return [pl.BlockSpec(s, m) for s, m in shapes_and_maps]


def _tc_stage1(x, p, Wc1, wcomb, cst):
    n, f = x.shape
    fo = Wc1.shape[1]
    grid = (n // _R,)

    def body(x_ref, p_ref, w_ref, wc_ref, c_ref, dis_ref, v_ref, *z_refs):
        xb = x_ref[...]
        deg = 1.0 + p_ref[0, :, 0] + p_ref[1, :, 0]
        d = lax.rsqrt(deg)
        dis_ref[...] = d[:, None]
        v_ref[...] = (jnp.sum(xb * wc_ref[0][None, :], axis=1) + c_ref[0, 0])[:, None]
        hw = jnp.dot(xb, w_ref[...], preferred_element_type=jnp.float32)
        z = hw * d[:, None]
        for c, zr in enumerate(z_refs):
            zr[...] = z[:, c * LANES:(c + 1) * LANES]

    nchunk = fo // LANES
    return pl.pallas_call(
        body,
        grid=grid,
        in_specs=_tc_specs([
            ((_R, f), lambda i: (i, 0)),
            ((2, _R, LANES), lambda i: (0, i, 0)),
            ((f, fo), lambda i: (0, 0)),
            ((1, f), lambda i: (0, 0)),
            ((1, 1), lambda i: (0, 0)),
        ]),
        out_specs=_tc_specs(
            [((_R, 1), lambda i: (i, 0))] * 2
            + [((_R, LANES), lambda i: (i, 0))] * nchunk),
        out_shape=[jax.ShapeDtypeStruct((n, 1), jnp.float32)] * 2
        + [jax.ShapeDtypeStruct((n, LANES), jnp.float32)] * nchunk,
    )(x, p, Wc1, wcomb, cst)


def _tc_stage_mid(acc, zs, dis, bc, Wnext, Mk, u_prev, partial_acc):
    n = zs[0].shape[0]
    nc = len(zs)
    na = acc.shape[0]
    fo = Wnext.shape[1]
    nco = fo // LANES
    grid = (n // _R,)
    nin = 1 + nc

    def body(*refs):
        acc_ref = refs[0]
        z_refs = refs[1:1 + nc]
        dis_ref, bc_ref, w_ref, m_ref, up_ref = refs[1 + nc:1 + nc + 5]
        out_refs = refs[1 + nc + 5:]
        zo_refs, u_ref = out_refs[:-1], out_refs[-1]
        d = dis_ref[...]
        u = up_ref[..., 0]
        znext = jnp.zeros((_R, fo), jnp.float32)
        for c in range(nc):
            if partial_acc:
                a = acc_ref[0] + acc_ref[1]
            else:
                a = acc_ref[c]
            h = jnp.maximum(
                d * (a + z_refs[c][...]) + bc_ref[0, c * LANES:(c + 1) * LANES][None, :],
                0.0)
            znext = znext + jnp.dot(
                h, w_ref[c * LANES:(c + 1) * LANES, :],
                preferred_element_type=jnp.float32)
            u = u + jnp.sum(h * m_ref[0, c * LANES:(c + 1) * LANES][None, :], axis=1)
        znext = znext * d
        for c in range(nco):
            zo_refs[c][...] = znext[:, c * LANES:(c + 1) * LANES]
        u_ref[...] = u[:, None]

    fi = nc * LANES
    return pl.pallas_call(
        body,
        grid=grid,
        in_specs=_tc_specs(
            [((na, _R, LANES), lambda i: (0, i, 0))]
            + [((_R, LANES), lambda i: (i, 0))] * nc
            + [((_R, 1), lambda i: (i, 0)),
               ((1, fi), lambda i: (0, 0)),
               ((fi, fo), lambda i: (0, 0)),
               ((1, fi), lambda i: (0, 0)),
               ((_R, 1), lambda i: (i, 0))]),
        out_specs=_tc_specs(
            [((_R, LANES), lambda i: (i, 0))] * nco
            + [((_R, 1), lambda i: (i, 0))]),
        out_shape=[jax.ShapeDtypeStruct((n, LANES), jnp.float32)] * nco
        + [jax.ShapeDtypeStruct((n, 1), jnp.float32)],
    )(acc, *zs, dis, bc, Wnext, Mk, u_prev)


def _tc_stage_last(acc, z4, dis, bc, Mk, u_prev):
    n = z4.shape[0]
    grid = (n // _R,)

    def body(acc_ref, z_ref, dis_ref, bc_ref, m_ref, up_ref, u_ref):
        d = dis_ref[...]
        a = acc_ref[0] + acc_ref[1]
        h = jnp.maximum(d * (a + z_ref[...]) + bc_ref[0][None, :], 0.0)
        u_ref[...] = up_ref[...] + jnp.sum(h * m_ref[0][None, :], axis=1)[:, None]

    return pl.pallas_call(
        body,
        grid=grid,
        in_specs=_tc_specs([
            ((2, _R, LANES), lambda i: (0, i, 0)),
            ((_R, LANES), lambda i: (i, 0)),
            ((_R, 1), lambda i: (i, 0)),
            ((1, LANES), lambda i: (0, 0)),
            ((1, LANES), lambda i: (0, 0)),
            ((_R, 1), lambda i: (i, 0)),
        ]),
        out_specs=pl.BlockSpec((_R, 1), lambda i: (i, 0)),
        out_shape=jax.ShapeDtypeStruct((n, 1), jnp.float32),
    )(acc, z4, dis, bc, Mk, u_prev)


def kernel(x, edge_index, Wc1, bc1, Wc2, bc2, Wc3, bc3, Wc4, bc4,
           Wl1, bl1, Wl2, bl2, Wl3, bl3, Wm1, bm1, Wm2, bm2, Wm3, bm3,
           Wm4, bm4, Wcls, bcls):
    n, f = x.shape
    e = edge_index.shape[1]
    n8 = (n // 128 + 1) * 128
    nblk = -(-e // (NTILE * EBLK))
    ep = NTILE * nblk * EBLK

    M4 = Wcls[:, 0]
    M3 = Wl3 @ M4
    M2 = Wl2 @ M3
    M1 = Wl1 @ M2
    wcomb = (Wm1 @ M1 + Wm2 @ M2 + Wm3 @ M3 + Wm4 @ M4)[None, :]
    cst = (bl1 @ M2 + bl2 @ M3 + bl3 @ M4
           + bm1 @ M1 + bm2 @ M2 + bm3 @ M3 + bm4 @ M4 + bcls[0])
    cst = jnp.asarray(cst, jnp.float32)[None, None]

    Wc3p = jnp.pad(Wc3, ((0, 0), (0, LANES - Wc3.shape[1])))
    bc3p = jnp.pad(bc3, (0, LANES - bc3.shape[0]))[None, :]
    Wc4p = jnp.pad(Wc4, ((0, LANES - Wc4.shape[0]), (0, LANES - Wc4.shape[1])))
    bc4p = jnp.pad(bc4, (0, LANES - bc4.shape[0]))[None, :]
    M1r, M2r = M1[None, :], M2[None, :]
    M3p = jnp.pad(M3, (0, LANES - M3.shape[0]))[None, :]
    M4p = jnp.pad(M4, (0, LANES - M4.shape[0]))[None, :]
    bc1r, bc2r = bc1[None, :], bc2[None, :]

    row = edge_index[0].astype(jnp.int32)
    col = edge_index[1].astype(jnp.int32)
    row_r = jnp.pad(row, (0, ep - e)).reshape(NTILE, nblk, EBLK)
    col_r = jnp.pad(col, (0, ep - e), constant_values=n).reshape(NTILE, nblk, EBLK)

    degp = _make_deg_kernel(n8, nblk)(col_r)

    dis, v, *z1 = _tc_stage1(x, degp, Wc1, wcomb, cst)
    acc1 = _make_scatter_kernel(4, n8, nblk)(*z1, row_r, col_r)
    z2_and_u = _tc_stage_mid(acc1, z1, dis, bc1r, Wc2, M1r, jnp.zeros((n, 1), jnp.float32), False)
    z2, u = z2_and_u[:-1], z2_and_u[-1]

    acc2 = _make_scatter_kernel(2, n8, nblk)(*z2, row_r, col_r)
    z3_and_u = _tc_stage_mid(acc2, z2, dis, bc2r, Wc3p, M2r, u, False)
    z3, u = z3_and_u[:-1], z3_and_u[-1]

    acc3 = _make_scatter_kernel(1, n8, nblk)(*z3, row_r, col_r)
    z4_and_u = _tc_stage_mid(acc3, z3, dis, bc3p, Wc4p, M3p, u, True)
    z4, u = z4_and_u[:-1], z4_and_u[-1]

    acc4 = _make_scatter_kernel(1, n8, nblk)(*z4, row_r, col_r)
    u = _tc_stage_last(acc4, z4[0], dis, bc4p, M4p, u)

    v_r = jnp.pad(v[:, 0], (0, ep - n)).reshape(NTILE, nblk, EBLK)
    out = _make_edge_out_kernel(n, n8, nblk)(u[:, 0], row_r, col_r, v_r)
    return out.reshape(-1)[:e]


# speedup vs baseline: 7.5309x; 1.6538x over previous
import functools

import jax
import jax.numpy as jnp
from jax import lax
from jax.experimental import pallas as pl
from jax.experimental.pallas import tpu as pltpu
from jax.experimental.pallas import tpu_sc as plsc

LANES = 16
NSUB = 16
NCORE = 2
NTILE = NCORE * NSUB
EBLK = 128
WIDE = 128

_mesh = lambda: plsc.VectorSubcoreMesh(
    core_axis_name="c", subcore_axis_name="s", num_cores=NCORE, num_subcores=NSUB)


def _fill(ref, nrows, value):
    val = jnp.full((LANES,), value, jnp.float32)

    def body(i, _):
        ref.at[i][...] = val
        return 0

    lax.fori_loop(0, nrows, body, 0)


def _zero_own_rows(acc, zbuf, sid, rows_per_sub):
    base = sid * rows_per_sub
    nfull = rows_per_sub // 1024
    rem = rows_per_sub - nfull * 1024
    for t in range(nfull):
        pltpu.sync_copy(zbuf, acc.at[pl.ds(base + t * 1024, 1024)])
    if rem:
        pltpu.sync_copy(zbuf.at[pl.ds(0, rem)], acc.at[pl.ds(base + nfull * 1024, rem)])


def _scale_idx(row_v, row8_v, nblk, chunk):
    def body(j, _):
        src_row = row_v.at[j]
        dst_row = row8_v.at[j]
        for t in range(EBLK // LANES):
            sl = pl.ds(t * LANES, LANES)
            dst_row[sl] = src_row[sl] * 8 + chunk
        return 0

    lax.fori_loop(0, nblk, body, 0)


def _scatter_chunk(z_ref, row8_v, col_v, gbuf, gsems, acc, nblk):
    pltpu.make_async_copy(z_ref.at[row8_v.at[0]], gbuf.at[0], gsems.at[0]).start()

    def step(j, _):
        b = lax.rem(j, 2)
        pb = 1 - b
        pltpu.make_async_copy(z_ref.at[row8_v.at[j]], gbuf.at[b], gsems.at[b]).start()
        pltpu.make_async_copy(z_ref.at[pl.ds(0, EBLK)], gbuf.at[pb], gsems.at[pb]).wait()
        pltpu.sync_copy(gbuf.at[pb], acc.at[col_v.at[j - 1]], add=True)
        return 0

    lax.fori_loop(1, nblk, step, 0)
    bl = (nblk - 1) % 2
    pltpu.make_async_copy(z_ref.at[pl.ds(0, EBLK)], gbuf.at[bl], gsems.at[bl]).wait()
    pltpu.sync_copy(gbuf.at[bl], acc.at[col_v.at[nblk - 1]], add=True)


def _writeback(acc, out_ref, col0, sid, rows_per_sub):
    base = sid * rows_per_sub
    pltpu.sync_copy(acc.at[pl.ds(base, rows_per_sub)],
                    out_ref.at[pl.ds(base, rows_per_sub), pl.ds(col0, LANES)])


def _make_deg_kernel(n_p, nblk):
    rows_per_sub = n_p // NSUB

    @functools.partial(
        pl.kernel,
        out_type=jax.ShapeDtypeStruct((n_p, WIDE), jnp.float32),
        mesh=_mesh(),
        compiler_params=pltpu.CompilerParams(use_tc_tiling_on_sc=False),
        scratch_types=[
            pltpu.VMEM((nblk, EBLK), jnp.int32),
            pltpu.VMEM((EBLK, LANES), jnp.float32),
            pltpu.VMEM((1024, LANES), jnp.float32),
            pltpu.VMEM_SHARED((n_p, LANES), jnp.float32),
        ],
    )
    def deg_kernel(col_hbm, out_ref, col_v, obuf, zbuf, acc):
        cid = lax.axis_index("c")
        sid = lax.axis_index("s")
        wid = sid * NCORE + cid
        pltpu.sync_copy(col_hbm.at[wid], col_v)
        _fill(obuf, EBLK, 1.0)
        _fill(zbuf, 1024, 0.0)
        _zero_own_rows(acc, zbuf, sid, rows_per_sub)
        plsc.subcore_barrier()

        def body(j, _):
            pltpu.sync_copy(obuf, acc.at[col_v.at[j]], add=True)
            return 0

        lax.fori_loop(0, nblk, body, 0)
        plsc.subcore_barrier()

        @pl.when(cid == 0)
        def _():
            _writeback(acc, out_ref, 0, sid, rows_per_sub)

        @pl.when(cid == 1)
        def _():
            _writeback(acc, out_ref, LANES, sid, rows_per_sub)

    return deg_kernel


def _make_scatter_kernel(nc, n_p, nblk):
    rows_per_sub = n_p // NSUB

    @functools.partial(
        pl.kernel,
        out_type=jax.ShapeDtypeStruct((n_p, WIDE), jnp.float32),
        mesh=_mesh(),
        compiler_params=pltpu.CompilerParams(use_tc_tiling_on_sc=False),
        scratch_types=[
            pltpu.VMEM((nblk, EBLK), jnp.int32),
            pltpu.VMEM((nblk, EBLK), jnp.int32),
            pltpu.VMEM((nblk, EBLK), jnp.int32),
            pltpu.VMEM((2, EBLK, LANES), jnp.float32),
            pltpu.VMEM((1024, LANES), jnp.float32),
            pltpu.VMEM_SHARED((n_p, LANES), jnp.float32),
            pltpu.SemaphoreType.DMA((2,)),
        ],
    )
    def scatter_kernel(z_hbm, row_hbm, col_hbm, out_ref,
                       row_v, row8_v, col_v, gbuf, zbuf, acc, gsems):
        zv = z_hbm
        cid = lax.axis_index("c")
        sid = lax.axis_index("s")
        _fill(zbuf, 1024, 0.0)
        wid = sid * NCORE + cid

        def load_idx(slot):
            pltpu.sync_copy(row_hbm.at[slot], row_v)
            pltpu.sync_copy(col_hbm.at[slot], col_v)

        def process(chunk, col0):
            _zero_own_rows(acc, zbuf, sid, rows_per_sub)
            plsc.subcore_barrier()
            if nc == 1:
                load_idx(wid)
                _scale_idx(row_v, row8_v, nblk, chunk)
                _scatter_chunk(zv, row8_v, col_v, gbuf, gsems, acc, nblk)
            else:
                for half in range(2):
                    load_idx(2 * sid + half)
                    _scale_idx(row_v, row8_v, nblk, chunk)
                    _scatter_chunk(zv, row8_v, col_v, gbuf, gsems, acc, nblk)
            plsc.subcore_barrier()
            _writeback(acc, out_ref, col0, sid, rows_per_sub)
            plsc.subcore_barrier()

        if nc == 4:
            @pl.when(cid == 0)
            def _():
                process(0, 0)
                process(1, LANES)

            @pl.when(cid == 1)
            def _():
                process(2, 2 * LANES)
                process(3, 3 * LANES)
        elif nc == 2:
            @pl.when(cid == 0)
            def _():
                process(0, 0)

            @pl.when(cid == 1)
            def _():
                process(1, LANES)
        else:
            @pl.when(cid == 0)
            def _():
                process(0, 0)

            @pl.when(cid == 1)
            def _():
                process(0, LANES)

    return scatter_kernel


def _make_edge_out_kernel(n_p, nblk):
    @functools.partial(
        pl.kernel,
        out_type=jax.ShapeDtypeStruct((NTILE, nblk, EBLK), jnp.float32),
        mesh=_mesh(),
        compiler_params=pltpu.CompilerParams(
            use_tc_tiling_on_sc=False, needs_layout_passes=False),
        scratch_types=[
            pltpu.VMEM((n_p,), jnp.float32),
            pltpu.VMEM((nblk, EBLK), jnp.int32),
            pltpu.VMEM((nblk, EBLK), jnp.int32),
            pltpu.VMEM((nblk, EBLK), jnp.float32),
            pltpu.VMEM((nblk, EBLK), jnp.float32),
        ],
    )
    def edge_out_kernel(u_hbm, row_hbm, col_hbm, v_hbm, out_ref,
                        u_v, row_v, col_v, v_v, out_v):
        cid = lax.axis_index("c")
        sid = lax.axis_index("s")
        wid = sid * NCORE + cid
        pltpu.sync_copy(u_hbm, u_v)
        pltpu.sync_copy(row_hbm.at[wid], row_v)
        pltpu.sync_copy(col_hbm.at[wid], col_v)
        pltpu.sync_copy(v_hbm.at[wid], v_v)

        def body(j, _):
            rr, cc = row_v.at[j], col_v.at[j]
            vv, oo = v_v.at[j], out_v.at[j]
            for t in range(EBLK // LANES):
                sl = pl.ds(t * LANES, LANES)
                g1 = plsc.load_gather(u_v, [rr[sl]])
                g2 = plsc.load_gather(u_v, [cc[sl]])
                oo[sl] = g1 + g2 + vv[sl]
            return 0

        lax.fori_loop(0, nblk, body, 0)
        pltpu.sync_copy(out_v, out_ref.at[wid])

    return edge_out_kernel


def _lane_mask(width):
    return (lax.broadcasted_iota(jnp.int32, (1, WIDE), 1) < width)


def _tc_specs(shapes_and_maps):
    return [pl.BlockSpec(s, m) for s, m in shapes_and_maps]


def _tc_stage1(x, p, Wc1p, wcomb, cst, n_p, r):
    grid = (n_p // r,)

    def body(x_ref, p_ref, w_ref, wc_ref, c_ref, dis_ref, z_ref, vp_ref):
        xb = x_ref[...]
        deg = 1.0 + p_ref[:, 0:1] + p_ref[:, LANES:LANES + 1]
        d = lax.rsqrt(deg)
        dis_ref[...] = d
        hw = jnp.dot(xb, w_ref[...], preferred_element_type=jnp.float32)
        z_ref[...] = hw * d
        v = jnp.sum(xb * wc_ref[0][None, :], axis=1) + c_ref[0, 0]
        vp_ref[...] = v[:, None]

    return pl.pallas_call(
        body,
        grid=grid,
        in_specs=_tc_specs([
            ((r, WIDE), lambda i: (i, 0)),
            ((r, WIDE), lambda i: (i, 0)),
            ((WIDE, WIDE), lambda i: (0, 0)),
            ((1, WIDE), lambda i: (0, 0)),
            ((1, 1), lambda i: (0, 0)),
        ]),
        out_specs=_tc_specs([
            ((r, 1), lambda i: (i, 0)),
            ((r, WIDE), lambda i: (i, 0)),
            ((r, 1), lambda i: (i, 0)),
        ]),
        out_shape=[
            jax.ShapeDtypeStruct((n_p, 1), jnp.float32),
            jax.ShapeDtypeStruct((n_p, WIDE), jnp.float32),
            jax.ShapeDtypeStruct((n_p, 1), jnp.float32),
        ],
    )(x, p, Wc1p, wcomb, cst)


def _tc_stage_mid(acc, z, dis, bc, Wnext, Mk, u_prev, fin, partial_acc, n_p, r):
    grid = (n_p // r,)

    def body(acc_ref, z_ref, dis_ref, bc_ref, w_ref, m_ref, up_ref,
             zo_ref, u_ref):
        d = dis_ref[...]
        if partial_acc:
            a = acc_ref[:, 0:LANES] + acc_ref[:, LANES:2 * LANES]
            zz = z_ref[:, 0:LANES]
            h = jnp.maximum(d * (a + zz) + bc_ref[0, 0:LANES][None, :], 0.0)
            h = jnp.where(_lane_mask(fin)[:, 0:LANES], h, 0.0)
            hw = jnp.dot(h, w_ref[0:LANES, :], preferred_element_type=jnp.float32)
            u_new = jnp.sum(h * m_ref[0, 0:LANES][None, :], axis=1)
        else:
            a = acc_ref[...]
            h = jnp.maximum(d * (a + z_ref[...]) + bc_ref[0][None, :], 0.0)
            h = jnp.where(_lane_mask(fin), h, 0.0)
            hw = jnp.dot(h, w_ref[...], preferred_element_type=jnp.float32)
            u_new = jnp.sum(h * m_ref[0][None, :], axis=1)
        zo_ref[...] = hw * d
        u_ref[...] = up_ref[...] + u_new[:, None]

    wpad = Wnext.shape[0]
    return pl.pallas_call(
        body,
        grid=grid,
        in_specs=_tc_specs([
            ((r, WIDE), lambda i: (i, 0)),
            ((r, WIDE), lambda i: (i, 0)),
            ((r, 1), lambda i: (i, 0)),
            ((1, WIDE), lambda i: (0, 0)),
            ((wpad, WIDE), lambda i: (0, 0)),
            ((1, WIDE), lambda i: (0, 0)),
            ((r, 1), lambda i: (i, 0)),
        ]),
        out_specs=_tc_specs([
            ((r, WIDE), lambda i: (i, 0)),
            ((r, 1), lambda i: (i, 0)),
        ]),
        out_shape=[
            jax.ShapeDtypeStruct((n_p, WIDE), jnp.float32),
            jax.ShapeDtypeStruct((n_p, 1), jnp.float32),
        ],
    )(acc, z, dis, bc, Wnext, Mk, u_prev)


def _tc_stage_last(acc, z4, dis, bc, Mk, u_prev, n_p, r):
    grid = (n_p // r,)

    def body(acc_ref, z_ref, dis_ref, bc_ref, m_ref, up_ref, u_ref):
        d = dis_ref[...]
        a = acc_ref[:, 0:LANES] + acc_ref[:, LANES:2 * LANES]
        h = jnp.maximum(d * (a + z_ref[:, 0:LANES]) + bc_ref[0, 0:LANES][None, :], 0.0)
        u = up_ref[..., 0] + jnp.sum(h * m_ref[0, 0:LANES][None, :], axis=1)
        u_ref[...] = u[:, None]

    return pl.pallas_call(
        body,
        grid=grid,
        in_specs=_tc_specs([
            ((r, WIDE), lambda i: (i, 0)),
            ((r, WIDE), lambda i: (i, 0)),
            ((r, 1), lambda i: (i, 0)),
            ((1, WIDE), lambda i: (0, 0)),
            ((1, WIDE), lambda i: (0, 0)),
            ((r, 1), lambda i: (i, 0)),
        ]),
        out_specs=pl.BlockSpec((r, 1), lambda i: (i, 0)),
        out_shape=jax.ShapeDtypeStruct((n_p, 1), jnp.float32),
    )(acc, z4, dis, bc, Mk, u_prev)


def _pad_w(w, rows, cols):
    return jnp.pad(w, ((0, rows - w.shape[0]), (0, cols - w.shape[1])))


def _pad_v(b, cols):
    return jnp.pad(b, (0, cols - b.shape[0]))[None, :]


def kernel(x, edge_index, Wc1, bc1, Wc2, bc2, Wc3, bc3, Wc4, bc4,
           Wl1, bl1, Wl2, bl2, Wl3, bl3, Wm1, bm1, Wm2, bm2, Wm3, bm3,
           Wm4, bm4, Wcls, bcls):
    n, f = x.shape
    e = edge_index.shape[1]
    n_p = (n // 128 + 1) * 128
    r = 4352
    nblk = -(-e // (NTILE * EBLK))
    ep = NTILE * nblk * EBLK

    M4 = Wcls[:, 0]
    M3 = Wl3 @ M4
    M2 = Wl2 @ M3
    M1 = Wl1 @ M2
    wcomb = (Wm1 @ M1 + Wm2 @ M2 + Wm3 @ M3 + Wm4 @ M4)[None, :]
    cst = (bl1 @ M2 + bl2 @ M3 + bl3 @ M4
           + bm1 @ M1 + bm2 @ M2 + bm3 @ M3 + bm4 @ M4 + bcls[0])
    cst = jnp.asarray(cst, jnp.float32)[None, None]

    Wc1p = _pad_w(Wc1, WIDE, WIDE)
    Wc2p = _pad_w(Wc2, WIDE, WIDE)
    Wc3p = _pad_w(Wc3, WIDE, WIDE)
    Wc4p = _pad_w(Wc4, LANES, WIDE)
    bc1p, bc2p, bc3p, bc4p = (_pad_v(b, WIDE) for b in (bc1, bc2, bc3, bc4))
    M1p, M2p, M3p, M4p = (_pad_v(m, WIDE) for m in (M1, M2, M3, M4))

    row = edge_index[0].astype(jnp.int32)
    col = edge_index[1].astype(jnp.int32)
    row_r = jnp.pad(row, (0, ep - e)).reshape(NTILE, nblk, EBLK)
    col_r = jnp.pad(col, (0, ep - e), constant_values=n).reshape(NTILE, nblk, EBLK)

    degp = _make_deg_kernel(n_p, nblk)(col_r)

    u0 = jnp.zeros((n_p, 1), jnp.float32)

    dis, z1, vp = _tc_stage1(x, degp, Wc1p, wcomb, cst, n_p, r)
    acc1 = _make_scatter_kernel(4, n_p, nblk)(z1.reshape(8 * n_p, LANES), row_r, col_r)
    z2, u = _tc_stage_mid(acc1, z1, dis, bc1p, Wc2p, M1p, u0, 64, False, n_p, r)

    acc2 = _make_scatter_kernel(2, n_p, nblk)(z2.reshape(8 * n_p, LANES), row_r, col_r)
    z3, u = _tc_stage_mid(acc2, z2, dis, bc2p, Wc3p, M2p, u, 32, False, n_p, r)

    acc3 = _make_scatter_kernel(1, n_p, nblk)(z3.reshape(8 * n_p, LANES), row_r, col_r)
    z4, u = _tc_stage_mid(acc3, z3, dis, bc3p, Wc4p, M3p, u, 8, True, n_p, r)

    acc4 = _make_scatter_kernel(1, n_p, nblk)(z4.reshape(8 * n_p, LANES), row_r, col_r)
    up = _tc_stage_last(acc4, z4, dis, bc4p, M4p, u, n_p, r)

    v_r = jnp.pad(vp[:, 0], (0, ep - n_p)).reshape(NTILE, nblk, EBLK)
    out = _make_edge_out_kernel(n_p, nblk)(up[:, 0], row_r, col_r, v_r)
    return out.reshape(-1)[:e]


# speedup vs baseline: 7.7992x; 1.0356x over previous
import functools

import jax
import jax.numpy as jnp
from jax import lax
from jax.experimental import pallas as pl
from jax.experimental.pallas import tpu as pltpu
from jax.experimental.pallas import tpu_sc as plsc

LANES = 16
NSUB = 16
NCORE = 2
NTILE = NCORE * NSUB
EBLK = 128
WIDE = 128

_mesh = lambda: plsc.VectorSubcoreMesh(
    core_axis_name="c", subcore_axis_name="s", num_cores=NCORE, num_subcores=NSUB)


def _fill(ref, nrows, value):
    val = jnp.full((LANES,), value, jnp.float32)

    def body(i, _):
        ref.at[i][...] = val
        return 0

    lax.fori_loop(0, nrows, body, 0)


def _zero_own_rows(acc, zbuf, sid, rows_per_sub):
    base = sid * rows_per_sub
    nfull = rows_per_sub // 1024
    rem = rows_per_sub - nfull * 1024
    for t in range(nfull):
        pltpu.sync_copy(zbuf, acc.at[pl.ds(base + t * 1024, 1024)])
    if rem:
        pltpu.sync_copy(zbuf.at[pl.ds(0, rem)], acc.at[pl.ds(base + nfull * 1024, rem)])


def _scale_idx(row_v, row8_v, nblk, chunk):
    def body(j, _):
        src_row = row_v.at[j]
        dst_row = row8_v.at[j]
        for t in range(EBLK // LANES):
            sl = pl.ds(t * LANES, LANES)
            dst_row[sl] = src_row[sl] * 8 + chunk
        return 0

    lax.fori_loop(0, nblk, body, 0)


NBUF = 2


def _scatter_chunk(z_ref, row8_v, col_v, gbuf, gsems, ssems, acc, nblk):
    def gwait(b):
        pltpu.make_async_copy(z_ref.at[pl.ds(0, EBLK)], gbuf.at[b], gsems.at[b]).wait()

    def swait(b):
        pltpu.make_async_copy(z_ref.at[pl.ds(0, EBLK)], gbuf.at[b], ssems.at[b]).wait()

    def step(j, _):
        b = lax.rem(j, NBUF)

        @pl.when(j >= NBUF)
        def _():
            swait(b)
        pltpu.make_async_copy(z_ref.at[row8_v.at[j]], gbuf.at[b], gsems.at[b]).start()

        @pl.when(j >= 1)
        def _():
            p = lax.rem(j - 1, NBUF)
            gwait(p)
            pltpu.async_copy(gbuf.at[p], acc.at[col_v.at[j - 1]], ssems.at[p], add=True)
        return 0

    lax.fori_loop(0, nblk, step, 0)
    last = lax.rem(nblk - 1, NBUF)
    gwait(last)
    pltpu.async_copy(gbuf.at[last], acc.at[col_v.at[nblk - 1]], ssems.at[last], add=True)
    for t in range(min(NBUF, nblk)):
        swait((nblk - 1 - t) % NBUF)


def _writeback(acc, out_ref, col0, sid, rows_per_sub):
    base = sid * rows_per_sub
    pltpu.sync_copy(acc.at[pl.ds(base, rows_per_sub)],
                    out_ref.at[pl.ds(base, rows_per_sub), pl.ds(col0, LANES)])


def _make_deg_kernel(n_p, nblk):
    rows_per_sub = n_p // NSUB

    @functools.partial(
        pl.kernel,
        out_type=jax.ShapeDtypeStruct((n_p, WIDE), jnp.float32),
        mesh=_mesh(),
        compiler_params=pltpu.CompilerParams(use_tc_tiling_on_sc=False),
        scratch_types=[
            pltpu.VMEM((nblk, EBLK), jnp.int32),
            pltpu.VMEM((EBLK, LANES), jnp.float32),
            pltpu.VMEM((1024, LANES), jnp.float32),
            pltpu.VMEM_SHARED((n_p, LANES), jnp.float32),
            pltpu.SemaphoreType.DMA,
        ],
    )
    def deg_kernel(col_hbm, out_ref, col_v, obuf, zbuf, acc, dsem):
        cid = lax.axis_index("c")
        sid = lax.axis_index("s")
        wid = sid * NCORE + cid
        pltpu.sync_copy(col_hbm.at[wid], col_v)
        _fill(obuf, EBLK, 1.0)
        _fill(zbuf, 1024, 0.0)
        _zero_own_rows(acc, zbuf, sid, rows_per_sub)
        plsc.subcore_barrier()

        def body(j, _):
            pltpu.async_copy(obuf, acc.at[col_v.at[j]], dsem, add=True)
            return 0

        lax.fori_loop(0, nblk, body, 0)

        def drain(j, _):
            dummy = out_ref.at[pl.ds(0, EBLK), pl.ds(0, LANES)]
            pltpu.make_async_copy(dummy, obuf, dsem).wait()
            return 0

        lax.fori_loop(0, nblk, drain, 0)
        plsc.subcore_barrier()

        @pl.when(cid == 0)
        def _():
            _writeback(acc, out_ref, 0, sid, rows_per_sub)

        @pl.when(cid == 1)
        def _():
            _writeback(acc, out_ref, LANES, sid, rows_per_sub)

    return deg_kernel


def _make_scatter_kernel(nc, n_p, nblk):
    rows_per_sub = n_p // NSUB

    @functools.partial(
        pl.kernel,
        out_type=jax.ShapeDtypeStruct((n_p, WIDE), jnp.float32),
        mesh=_mesh(),
        compiler_params=pltpu.CompilerParams(use_tc_tiling_on_sc=False),
        scratch_types=[
            pltpu.VMEM((nblk, EBLK), jnp.int32),
            pltpu.VMEM((nblk, EBLK), jnp.int32),
            pltpu.VMEM((nblk, EBLK), jnp.int32),
            pltpu.VMEM((NBUF, EBLK, LANES), jnp.float32),
            pltpu.VMEM((1024, LANES), jnp.float32),
            pltpu.VMEM_SHARED((n_p, LANES), jnp.float32),
            pltpu.SemaphoreType.DMA((NBUF,)),
            pltpu.SemaphoreType.DMA((NBUF,)),
        ],
    )
    def scatter_kernel(z_hbm, row_hbm, col_hbm, out_ref,
                       row_v, row8_v, col_v, gbuf, zbuf, acc, gsems, ssems):
        zv = z_hbm
        cid = lax.axis_index("c")
        sid = lax.axis_index("s")
        _fill(zbuf, 1024, 0.0)
        wid = sid * NCORE + cid

        def load_idx(slot):
            pltpu.sync_copy(row_hbm.at[slot], row_v)
            pltpu.sync_copy(col_hbm.at[slot], col_v)

        def process(chunk, col0):
            _zero_own_rows(acc, zbuf, sid, rows_per_sub)
            plsc.subcore_barrier()
            if nc == 1:
                load_idx(wid)
                _scale_idx(row_v, row8_v, nblk, chunk)
                _scatter_chunk(zv, row8_v, col_v, gbuf, gsems, ssems, acc, nblk)
            else:
                for half in range(2):
                    load_idx(2 * sid + half)
                    _scale_idx(row_v, row8_v, nblk, chunk)
                    _scatter_chunk(zv, row8_v, col_v, gbuf, gsems, ssems, acc, nblk)
            plsc.subcore_barrier()
            _writeback(acc, out_ref, col0, sid, rows_per_sub)
            plsc.subcore_barrier()

        if nc == 4:
            @pl.when(cid == 0)
            def _():
                process(0, 0)
                process(1, LANES)

            @pl.when(cid == 1)
            def _():
                process(2, 2 * LANES)
                process(3, 3 * LANES)
        elif nc == 2:
            @pl.when(cid == 0)
            def _():
                process(0, 0)

            @pl.when(cid == 1)
            def _():
                process(1, LANES)
        else:
            @pl.when(cid == 0)
            def _():
                process(0, 0)

            @pl.when(cid == 1)
            def _():
                process(0, LANES)

    return scatter_kernel


def _make_edge_out_kernel(n_p, nblk):
    @functools.partial(
        pl.kernel,
        out_type=jax.ShapeDtypeStruct((NTILE, nblk, EBLK), jnp.float32),
        mesh=_mesh(),
        compiler_params=pltpu.CompilerParams(
            use_tc_tiling_on_sc=False, needs_layout_passes=False),
        scratch_types=[
            pltpu.VMEM((n_p,), jnp.float32),
            pltpu.VMEM((nblk, EBLK), jnp.int32),
            pltpu.VMEM((nblk, EBLK), jnp.int32),
            pltpu.VMEM((nblk, EBLK), jnp.float32),
            pltpu.VMEM((nblk, EBLK), jnp.float32),
        ],
    )
    def edge_out_kernel(u_hbm, row_hbm, col_hbm, v_hbm, out_ref,
                        u_v, row_v, col_v, v_v, out_v):
        cid = lax.axis_index("c")
        sid = lax.axis_index("s")
        wid = sid * NCORE + cid
        pltpu.sync_copy(u_hbm, u_v)
        pltpu.sync_copy(row_hbm.at[wid], row_v)
        pltpu.sync_copy(col_hbm.at[wid], col_v)
        pltpu.sync_copy(v_hbm.at[wid], v_v)

        def body(j, _):
            rr, cc = row_v.at[j], col_v.at[j]
            vv, oo = v_v.at[j], out_v.at[j]
            for t in range(EBLK // LANES):
                sl = pl.ds(t * LANES, LANES)
                g1 = plsc.load_gather(u_v, [rr[sl]])
                g2 = plsc.load_gather(u_v, [cc[sl]])
                oo[sl] = g1 + g2 + vv[sl]
            return 0

        lax.fori_loop(0, nblk, body, 0)
        pltpu.sync_copy(out_v, out_ref.at[wid])

    return edge_out_kernel


def _lane_mask(width):
    return (lax.broadcasted_iota(jnp.int32, (1, WIDE), 1) < width)


def _tc_specs(shapes_and_maps):
    return [pl.BlockSpec(s, m) for s, m in shapes_and_maps]


def _tc_stage1(x, p, Wc1p, wcomb, cst, n_p, r):
    grid = (n_p // r,)

    def body(x_ref, p_ref, w_ref, wc_ref, c_ref, dis_ref, z_ref, vp_ref):
        xb = x_ref[...]
        deg = 1.0 + p_ref[:, 0:1] + p_ref[:, LANES:LANES + 1]
        d = lax.rsqrt(deg)
        dis_ref[...] = d
        hw = jnp.dot(xb, w_ref[...], preferred_element_type=jnp.float32)
        z_ref[...] = hw * d
        vp_ref[...] = (jnp.sum(xb * wc_ref[0][None, :], axis=1) + c_ref[0, 0])[:, None]

    return pl.pallas_call(
        body,
        grid=grid,
        in_specs=_tc_specs([
            ((r, WIDE), lambda i: (i, 0)),
            ((r, WIDE), lambda i: (i, 0)),
            ((WIDE, WIDE), lambda i: (0, 0)),
            ((1, WIDE), lambda i: (0, 0)),
            ((1, 1), lambda i: (0, 0)),
        ]),
        out_specs=_tc_specs([
            ((r, 1), lambda i: (i, 0)),
            ((r, WIDE), lambda i: (i, 0)),
            ((r, 1), lambda i: (i, 0)),
        ]),
        out_shape=[
            jax.ShapeDtypeStruct((n_p, 1), jnp.float32),
            jax.ShapeDtypeStruct((n_p, WIDE), jnp.float32),
            jax.ShapeDtypeStruct((n_p, 1), jnp.float32),
        ],
    )(x, p, Wc1p, wcomb, cst)


def _tc_stage_mid(acc, z, dis, bc, Wnext, Mk, u_prev, fin, partial_acc, n_p, r):
    first = u_prev is None
    grid = (n_p // r,)

    def body(*refs):
        if first:
            (acc_ref, z_ref, dis_ref, bc_ref, w_ref, m_ref,
             zo_ref, u_ref) = refs
            up_ref = None
        else:
            (acc_ref, z_ref, dis_ref, bc_ref, w_ref, m_ref, up_ref,
             zo_ref, u_ref) = refs
        d = dis_ref[...]
        if partial_acc:
            a = acc_ref[:, 0:LANES] + acc_ref[:, LANES:2 * LANES]
            zz = z_ref[:, 0:LANES]
            h = jnp.maximum(d * (a + zz) + bc_ref[0, 0:LANES][None, :], 0.0)
            h = jnp.where(_lane_mask(fin)[:, 0:LANES], h, 0.0)
            hw = jnp.dot(h, w_ref[0:LANES, :], preferred_element_type=jnp.float32)
            u_new = jnp.sum(h * m_ref[0, 0:LANES][None, :], axis=1)
        else:
            a = acc_ref[...]
            h = jnp.maximum(d * (a + z_ref[...]) + bc_ref[0][None, :], 0.0)
            h = jnp.where(_lane_mask(fin), h, 0.0)
            hw = jnp.dot(h, w_ref[...], preferred_element_type=jnp.float32)
            u_new = jnp.sum(h * m_ref[0][None, :], axis=1)
        zo_ref[...] = hw * d
        if first:
            u_ref[...] = u_new[:, None]
        else:
            u_ref[...] = up_ref[...] + u_new[:, None]

    wpad = Wnext.shape[0]
    return pl.pallas_call(
        body,
        grid=grid,
        in_specs=_tc_specs([
            ((r, WIDE), lambda i: (i, 0)),
            ((r, WIDE), lambda i: (i, 0)),
            ((r, 1), lambda i: (i, 0)),
            ((1, WIDE), lambda i: (0, 0)),
            ((wpad, WIDE), lambda i: (0, 0)),
            ((1, WIDE), lambda i: (0, 0)),
        ] + ([] if first else [((r, 1), lambda i: (i, 0))])),
        out_specs=_tc_specs([
            ((r, WIDE), lambda i: (i, 0)),
            ((r, 1), lambda i: (i, 0)),
        ]),
        out_shape=[
            jax.ShapeDtypeStruct((n_p, WIDE), jnp.float32),
            jax.ShapeDtypeStruct((n_p, 1), jnp.float32),
        ],
    )(acc, z, dis, bc, Wnext, Mk,
      *([] if first else [u_prev]))


def _tc_stage_last(acc, z4, dis, bc, Mk, u_prev, n_p, r):
    grid = (n_p // r,)

    def body(acc_ref, z_ref, dis_ref, bc_ref, m_ref, up_ref, u_ref):
        d = dis_ref[...]
        a = acc_ref[:, 0:LANES] + acc_ref[:, LANES:2 * LANES]
        h = jnp.maximum(d * (a + z_ref[:, 0:LANES]) + bc_ref[0, 0:LANES][None, :], 0.0)
        u_ref[...] = up_ref[...] + jnp.sum(h * m_ref[0, 0:LANES][None, :], axis=1)[:, None]

    return pl.pallas_call(
        body,
        grid=grid,
        in_specs=_tc_specs([
            ((r, WIDE), lambda i: (i, 0)),
            ((r, WIDE), lambda i: (i, 0)),
            ((r, 1), lambda i: (i, 0)),
            ((1, WIDE), lambda i: (0, 0)),
            ((1, WIDE), lambda i: (0, 0)),
            ((r, 1), lambda i: (i, 0)),
        ]),
        out_specs=pl.BlockSpec((r, 1), lambda i: (i, 0)),
        out_shape=jax.ShapeDtypeStruct((n_p, 1), jnp.float32),
    )(acc, z4, dis, bc, Mk, u_prev)


def _pad_w(w, rows, cols):
    return jnp.pad(w, ((0, rows - w.shape[0]), (0, cols - w.shape[1])))


def _pad_v(b, cols):
    return jnp.pad(b, (0, cols - b.shape[0]))[None, :]


def kernel(x, edge_index, Wc1, bc1, Wc2, bc2, Wc3, bc3, Wc4, bc4,
           Wl1, bl1, Wl2, bl2, Wl3, bl3, Wm1, bm1, Wm2, bm2, Wm3, bm3,
           Wm4, bm4, Wcls, bcls):
    n, f = x.shape
    e = edge_index.shape[1]
    n_p = (n // 128 + 1) * 128
    r = 4352
    nblk = -(-e // (NTILE * EBLK))
    ep = NTILE * nblk * EBLK

    M4 = Wcls[:, 0]
    M3 = Wl3 @ M4
    M2 = Wl2 @ M3
    M1 = Wl1 @ M2
    wcomb = (Wm1 @ M1 + Wm2 @ M2 + Wm3 @ M3 + Wm4 @ M4)[None, :]
    cst = (bl1 @ M2 + bl2 @ M3 + bl3 @ M4
           + bm1 @ M1 + bm2 @ M2 + bm3 @ M3 + bm4 @ M4 + bcls[0])
    cst = jnp.asarray(cst, jnp.float32)[None, None]

    Wc1p = _pad_w(Wc1, WIDE, WIDE)
    Wc2p = _pad_w(Wc2, WIDE, WIDE)
    Wc3p = _pad_w(Wc3, WIDE, WIDE)
    Wc4p = _pad_w(Wc4, LANES, WIDE)
    bc1p, bc2p, bc3p, bc4p = (_pad_v(b, WIDE) for b in (bc1, bc2, bc3, bc4))
    M1p, M2p, M3p, M4p = (_pad_v(m, WIDE) for m in (M1, M2, M3, M4))

    row = edge_index[0].astype(jnp.int32)
    col = edge_index[1].astype(jnp.int32)
    row_r = jnp.pad(row, (0, ep - e)).reshape(NTILE, nblk, EBLK)
    col_r = jnp.pad(col, (0, ep - e), constant_values=n).reshape(NTILE, nblk, EBLK)

    degp = _make_deg_kernel(n_p, nblk)(col_r)

    dis, z1, vp = _tc_stage1(x, degp, Wc1p, wcomb, cst, n_p, r)
    acc1 = _make_scatter_kernel(4, n_p, nblk)(z1.reshape(8 * n_p, LANES), row_r, col_r)
    z2, u = _tc_stage_mid(acc1, z1, dis, bc1p, Wc2p, M1p, None, 64, False, n_p, r)

    acc2 = _make_scatter_kernel(2, n_p, nblk)(z2.reshape(8 * n_p, LANES), row_r, col_r)
    z3, u = _tc_stage_mid(acc2, z2, dis, bc2p, Wc3p, M2p, u, 32, False, n_p, r)

    acc3 = _make_scatter_kernel(1, n_p, nblk)(z3.reshape(8 * n_p, LANES), row_r, col_r)
    z4, u = _tc_stage_mid(acc3, z3, dis, bc3p, Wc4p, M3p, u, 8, True, n_p, r)

    acc4 = _make_scatter_kernel(1, n_p, nblk)(z4.reshape(8 * n_p, LANES), row_r, col_r)
    up = _tc_stage_last(acc4, z4, dis, bc4p, M4p, u, n_p, r)

    v_r = jnp.pad(vp[:, 0], (0, ep - n_p)).reshape(NTILE, nblk, EBLK)
    out = _make_edge_out_kernel(n_p, nblk)(up[:, 0], row_r, col_r, v_r)
    return out.reshape(-1)[:e]
